# baseline (speedup 1.0000x reference)
import functools

import jax
import jax.numpy as jnp
from jax import lax
from jax.experimental import pallas as pl
from jax.experimental.pallas import tpu as pltpu
from jax.experimental.pallas import tpu_sc as plsc


TILE = 128
_SC_MESH = dict(core_axis_name="c", subcore_axis_name="s")


def _sc_worker_id():
    return lax.axis_index("s") * 2 + lax.axis_index("c")


def _sc_dispatch(x_flat, wts_rep, out_pos, n_slots):
    n, h = x_flat.shape
    nw = 32
    chunk = n // nw

    @functools.partial(
        pl.kernel,
        mesh=plsc.VectorSubcoreMesh(**_SC_MESH),
        out_type=(
            jax.ShapeDtypeStruct((n_slots, h), x_flat.dtype),
            jax.ShapeDtypeStruct((n_slots, 128), wts_rep.dtype),
        ),
        scratch_types=[
            pltpu.VMEM((chunk,), jnp.int32),
            pltpu.VMEM((chunk, h), x_flat.dtype),
            pltpu.VMEM((chunk, 128), wts_rep.dtype),
        ],
    )
    def k(x_hbm, w_hbm, pos_hbm, xs_hbm, ws_hbm, pos_v, rows_v, wrow_v):
        base = _sc_worker_id() * chunk
        pltpu.sync_copy(pos_hbm.at[pl.ds(base, chunk)], pos_v)
        pltpu.sync_copy(x_hbm.at[pl.ds(base, chunk)], rows_v)
        pltpu.sync_copy(w_hbm.at[pl.ds(base, chunk)], wrow_v)
        pltpu.sync_copy(rows_v, xs_hbm.at[pos_v])
        pltpu.sync_copy(wrow_v, ws_hbm.at[pos_v])

    return k(x_flat, wts_rep, out_pos)


def _sc_combine(y_slots, out_pos, n):
    _, h = y_slots.shape
    nw = 32
    chunk = n // nw

    @functools.partial(
        pl.kernel,
        mesh=plsc.VectorSubcoreMesh(**_SC_MESH),
        out_type=jax.ShapeDtypeStruct((n, h), y_slots.dtype),
        scratch_types=[
            pltpu.VMEM((chunk,), jnp.int32),
            pltpu.VMEM((chunk, h), y_slots.dtype),
        ],
    )
    def k(ys_hbm, pos_hbm, out_hbm, pos_v, rows_v):
        base = _sc_worker_id() * chunk
        pltpu.sync_copy(pos_hbm.at[pl.ds(base, chunk)], pos_v)
        pltpu.sync_copy(ys_hbm.at[pos_v], rows_v)
        pltpu.sync_copy(rows_v, out_hbm.at[pl.ds(base, chunk)])

    return k(y_slots, out_pos)


FF_SPLIT = 4


def _ffn_tile_kernel(meta_ref, x_ref, w_ref, gate_ref, up_ref, down_ref,
                     out_ref, acc_ref, *, max_tiles):
    j = pl.program_id(0)
    c = pl.program_id(1)
    num_tiles = meta_ref[max_tiles]

    @pl.when(j < num_tiles)
    def _():
        x = x_ref[...]
        g = jax.lax.dot_general(
            x, gate_ref[0], (((1,), (1,)), ((), ())),
            preferred_element_type=jnp.float32)
        u = jax.lax.dot_general(
            x, up_ref[0], (((1,), (1,)), ((), ())),
            preferred_element_type=jnp.float32)
        inter = g * jax.nn.sigmoid(g) * u
        y = jax.lax.dot_general(
            inter, down_ref[0], (((1,), (1,)), ((), ())),
            preferred_element_type=jnp.float32)

        @pl.when(c == 0)
        def _():
            acc_ref[...] = y

        @pl.when(c > 0)
        def _():
            acc_ref[...] += y

        @pl.when(c == FF_SPLIT - 1)
        def _():
            out_ref[...] = acc_ref[...] * w_ref[:, :1]


def _grouped_ffn(x_slots, w_slots, meta, gate_proj, up_proj, down_proj,
                 max_tiles):
    n_slots, h = x_slots.shape
    e, ff, _ = gate_proj.shape
    ffc = ff // FF_SPLIT
    return pl.pallas_call(
        functools.partial(_ffn_tile_kernel, max_tiles=max_tiles),
        grid_spec=pltpu.PrefetchScalarGridSpec(
            num_scalar_prefetch=1,
            grid=(max_tiles, FF_SPLIT),
            in_specs=[
                pl.BlockSpec((TILE, h), lambda j, c, m: (j, 0)),
                pl.BlockSpec((TILE, 128), lambda j, c, m: (j, 0)),
                pl.BlockSpec((1, ffc, h), lambda j, c, m: (m[j], c, 0)),
                pl.BlockSpec((1, ffc, h), lambda j, c, m: (m[j], c, 0)),
                pl.BlockSpec((1, h, ffc), lambda j, c, m: (m[j], 0, c)),
            ],
            out_specs=pl.BlockSpec((TILE, h), lambda j, c, m: (j, 0)),
            scratch_shapes=[pltpu.VMEM((TILE, h), jnp.float32)],
        ),
        out_shape=jax.ShapeDtypeStruct((n_slots, h), jnp.float32),
        compiler_params=pltpu.CompilerParams(
            dimension_semantics=("arbitrary", "arbitrary")),
    )(meta, x_slots, w_slots, gate_proj, up_proj, down_proj)


def kernel(x, expert_indices, expert_weights, gate_proj, up_proj, down_proj):
    b, s, h = x.shape
    e, ff, _ = gate_proj.shape
    topk = expert_indices.shape[-1]
    n = b * s * topk
    max_tiles = n // TILE + e
    n_slots = max_tiles * TILE

    x_flat = x.reshape(-1, h)
    idx_flat = expert_indices.reshape(-1).astype(jnp.int32)
    wts_flat = expert_weights.reshape(-1)

    eids = jnp.arange(e, dtype=jnp.int32)
    onehot = (idx_flat[:, None] == eids[None, :]).astype(jnp.int32)
    cum = jnp.cumsum(onehot, axis=0)
    counts = cum[-1]
    rank = jnp.sum(cum * onehot, axis=1) - 1
    starts = jnp.cumsum(counts) - counts
    ends = starts + counts
    tiles_per_e = (counts + TILE - 1) // TILE
    tile_cum = jnp.cumsum(tiles_per_e)
    tile_off = tile_cum - tiles_per_e
    num_tiles = tile_cum[-1]

    def pick(vec):
        return jnp.sum(onehot * vec[None, :], axis=1)

    starts_t = pick(starts)
    ends_t = pick(ends)
    tile_off_t = pick(tile_off)
    kk = rank // TILE
    jj = tile_off_t + kk
    wstart_t = jnp.maximum(0, jnp.minimum(starts_t + kk * TILE, ends_t - TILE))
    out_pos = (jj * TILE + starts_t + rank - wstart_t).astype(jnp.int32)

    tile_ids = jnp.arange(max_tiles, dtype=jnp.int32)
    te = jnp.sum((tile_cum[None, :] <= tile_ids[:, None]).astype(jnp.int32),
                 axis=1)
    te_c = jnp.minimum(te, e - 1)
    last_e = jnp.max(jnp.where(counts > 0, eids, -1))
    tile_expert = jnp.where(tile_ids < num_tiles, te_c, last_e)
    meta = jnp.concatenate(
        [tile_expert, num_tiles[None]]).astype(jnp.int32)

    wts_rep = jnp.broadcast_to(wts_flat[:, None], (n, 128))
    x_slots, w_slots16 = _sc_dispatch(x_flat, wts_rep, out_pos, n_slots)

    y_slots = _grouped_ffn(x_slots, w_slots16, meta, gate_proj, up_proj,
                           down_proj, max_tiles)

    gathered = _sc_combine(y_slots, out_pos, n)
    if topk > 1:
        gathered = gathered.reshape(b * s, topk, h).sum(axis=1)
    return gathered.reshape(b, s, h)


# speedup vs baseline: 1.6442x; 1.6442x over previous
import functools

import jax
import jax.numpy as jnp
from jax import lax
from jax.experimental import pallas as pl
from jax.experimental.pallas import tpu as pltpu
from jax.experimental.pallas import tpu_sc as plsc


TILE = 128
_SC_MESH = dict(core_axis_name="c", subcore_axis_name="s")


def _sc_worker_id():
    return lax.axis_index("s") * 2 + lax.axis_index("c")


def _sc_dispatch(x_flat, wts_rep, out_pos, n_slots):
    n, h = x_flat.shape
    nw = 32
    chunk = n // nw

    @functools.partial(
        pl.kernel,
        mesh=plsc.VectorSubcoreMesh(**_SC_MESH),
        out_type=(
            jax.ShapeDtypeStruct((n_slots, h), x_flat.dtype),
            jax.ShapeDtypeStruct((n_slots, 128), wts_rep.dtype),
        ),
        scratch_types=[
            pltpu.VMEM((chunk,), jnp.int32),
            pltpu.VMEM((chunk, h), x_flat.dtype),
            pltpu.VMEM((chunk, 128), wts_rep.dtype),
        ],
    )
    def k(x_hbm, w_hbm, pos_hbm, xs_hbm, ws_hbm, pos_v, rows_v, wrow_v):
        base = _sc_worker_id() * chunk
        pltpu.sync_copy(pos_hbm.at[pl.ds(base, chunk)], pos_v)
        pltpu.sync_copy(x_hbm.at[pl.ds(base, chunk)], rows_v)
        pltpu.sync_copy(w_hbm.at[pl.ds(base, chunk)], wrow_v)
        pltpu.sync_copy(rows_v, xs_hbm.at[pos_v])
        pltpu.sync_copy(wrow_v, ws_hbm.at[pos_v])

    return k(x_flat, wts_rep, out_pos)


def _sc_combine(y_slots, out_pos, n):
    _, h = y_slots.shape
    nw = 32
    chunk = n // nw

    @functools.partial(
        pl.kernel,
        mesh=plsc.VectorSubcoreMesh(**_SC_MESH),
        out_type=jax.ShapeDtypeStruct((n, h), y_slots.dtype),
        scratch_types=[
            pltpu.VMEM((chunk,), jnp.int32),
            pltpu.VMEM((chunk, h), y_slots.dtype),
        ],
    )
    def k(ys_hbm, pos_hbm, out_hbm, pos_v, rows_v):
        base = _sc_worker_id() * chunk
        pltpu.sync_copy(pos_hbm.at[pl.ds(base, chunk)], pos_v)
        pltpu.sync_copy(ys_hbm.at[pos_v], rows_v)
        pltpu.sync_copy(rows_v, out_hbm.at[pl.ds(base, chunk)])

    return k(y_slots, out_pos)


def _ffn_tile_kernel(meta_ref, x_ref, w_ref, gate_ref, up_ref, down_ref,
                     out_ref, *, max_tiles):
    j = pl.program_id(0)
    num_tiles = meta_ref[max_tiles]

    @pl.when(j < num_tiles)
    def _():
        x = x_ref[...]
        g = jax.lax.dot_general(
            x, gate_ref[0], (((1,), (1,)), ((), ())),
            preferred_element_type=jnp.float32)
        u = jax.lax.dot_general(
            x, up_ref[0], (((1,), (1,)), ((), ())),
            preferred_element_type=jnp.float32)
        inter = g * jax.nn.sigmoid(g) * u
        y = jax.lax.dot_general(
            inter, down_ref[0], (((1,), (1,)), ((), ())),
            preferred_element_type=jnp.float32)
        out_ref[...] = y * w_ref[:, :1]


def _grouped_ffn(x_slots, w_slots, meta, gate_proj, up_proj, down_proj,
                 max_tiles):
    n_slots, h = x_slots.shape
    e, ff, _ = gate_proj.shape
    return pl.pallas_call(
        functools.partial(_ffn_tile_kernel, max_tiles=max_tiles),
        grid_spec=pltpu.PrefetchScalarGridSpec(
            num_scalar_prefetch=1,
            grid=(max_tiles,),
            in_specs=[
                pl.BlockSpec((TILE, h), lambda j, m: (j, 0)),
                pl.BlockSpec((TILE, 128), lambda j, m: (j, 0)),
                pl.BlockSpec((1, ff, h), lambda j, m: (m[j], 0, 0)),
                pl.BlockSpec((1, ff, h), lambda j, m: (m[j], 0, 0)),
                pl.BlockSpec((1, h, ff), lambda j, m: (m[j], 0, 0)),
            ],
            out_specs=pl.BlockSpec((TILE, h), lambda j, m: (j, 0)),
        ),
        out_shape=jax.ShapeDtypeStruct((n_slots, h), jnp.float32),
        compiler_params=pltpu.CompilerParams(
            dimension_semantics=("arbitrary",)),
    )(meta, x_slots, w_slots, gate_proj, up_proj, down_proj)


def kernel(x, expert_indices, expert_weights, gate_proj, up_proj, down_proj):
    b, s, h = x.shape
    e, ff, _ = gate_proj.shape
    topk = expert_indices.shape[-1]
    n = b * s * topk
    max_tiles = n // TILE + e
    n_slots = max_tiles * TILE

    x_flat = x.reshape(-1, h)
    idx_flat = expert_indices.reshape(-1).astype(jnp.int32)
    wts_flat = expert_weights.reshape(-1)

    eids = jnp.arange(e, dtype=jnp.int32)
    onehot = (idx_flat[:, None] == eids[None, :]).astype(jnp.int32)
    rows = jnp.arange(n, dtype=jnp.int32)
    tri = (rows[:, None] >= rows[None, :]).astype(jnp.bfloat16)
    cum = jax.lax.dot_general(
        tri, onehot.astype(jnp.bfloat16), (((1,), (0,)), ((), ())),
        preferred_element_type=jnp.float32)
    counts = cum[-1].astype(jnp.int32)
    rank = (jnp.sum(cum * onehot, axis=1) - 1).astype(jnp.int32)
    starts = jnp.cumsum(counts) - counts
    ends = starts + counts
    tiles_per_e = (counts + TILE - 1) // TILE
    tile_cum = jnp.cumsum(tiles_per_e)
    tile_off = tile_cum - tiles_per_e
    num_tiles = tile_cum[-1]

    def pick(vec):
        return jnp.sum(onehot * vec[None, :], axis=1)

    starts_t = pick(starts)
    ends_t = pick(ends)
    tile_off_t = pick(tile_off)
    kk = rank // TILE
    jj = tile_off_t + kk
    wstart_t = jnp.maximum(0, jnp.minimum(starts_t + kk * TILE, ends_t - TILE))
    out_pos = (jj * TILE + starts_t + rank - wstart_t).astype(jnp.int32)

    tile_ids = jnp.arange(max_tiles, dtype=jnp.int32)
    te = jnp.sum((tile_cum[None, :] <= tile_ids[:, None]).astype(jnp.int32),
                 axis=1)
    te_c = jnp.minimum(te, e - 1)
    last_e = jnp.max(jnp.where(counts > 0, eids, -1))
    tile_expert = jnp.where(tile_ids < num_tiles, te_c, last_e)
    meta = jnp.concatenate(
        [tile_expert, num_tiles[None]]).astype(jnp.int32)

    wts_rep = jnp.broadcast_to(wts_flat[:, None], (n, 128))
    x_slots, w_slots16 = _sc_dispatch(x_flat, wts_rep, out_pos, n_slots)

    y_slots = _grouped_ffn(x_slots, w_slots16, meta, gate_proj, up_proj,
                           down_proj, max_tiles)

    gathered = _sc_combine(y_slots, out_pos, n)
    if topk > 1:
        gathered = gathered.reshape(b * s, topk, h).sum(axis=1)
    return gathered.reshape(b, s, h)
